# baseline (speedup 1.0000x reference)
import functools

import jax
import jax.numpy as jnp
import numpy as np
from jax import lax
from jax.experimental import pallas as pl
from jax.experimental.pallas import tpu as pltpu
from jax.experimental.pallas import tpu_sc as plsc

_NUM_SAMPLES = 16


def _sample_indices(t: int) -> np.ndarray:
    stop = np.float32(t - 1)
    frac = np.arange(_NUM_SAMPLES - 1, dtype=np.float32) / np.float32(
        _NUM_SAMPLES - 1
    )
    vals = np.concatenate([stop * frac, np.array([stop], np.float32)])
    vals = np.clip(vals, np.float32(0.0), stop)
    return vals.astype(np.int32)


def kernel(x):
    t, c, hh, ww = x.shape
    idx = _sample_indices(t)
    info = plsc.get_sparse_core_info()
    nw = info.num_cores * info.num_subcores
    halves = nw // _NUM_SAMPLES
    hrows = hh // halves
    k = 2
    piece = hrows // k
    assert hh % halves == 0 and hrows % k == 0 and piece % 8 == 0

    assert all(int(idx[r]) == (r * (t - 1)) // (_NUM_SAMPLES - 1)
               for r in range(_NUM_SAMPLES))

    mesh = plsc.VectorSubcoreMesh(core_axis_name="c", subcore_axis_name="s")

    @functools.partial(
        pl.kernel,
        mesh=mesh,
        compiler_params=pltpu.CompilerParams(use_tc_tiling_on_sc=True),
        out_type=jax.ShapeDtypeStruct((_NUM_SAMPLES, c, hh, ww), jnp.float32),
        scratch_types=[
            pltpu.VMEM((c, piece, ww), jnp.float32),
            pltpu.VMEM((c, piece, ww), jnp.float32),
            pltpu.SemaphoreType.DMA,
            pltpu.SemaphoreType.DMA,
            pltpu.SemaphoreType.DMA,
            pltpu.SemaphoreType.DMA,
        ],
    )
    def gather_kernel(x_hbm, out_hbm, buf0, buf1, isem0, isem1, osem0, osem1):
        wid = lax.axis_index("s") * info.num_cores + lax.axis_index("c")
        r = wid // halves
        h = wid % halves
        src = (r * (t - 1)) // (_NUM_SAMPLES - 1)
        base = h * hrows
        bufs = (buf0, buf1)
        isems = (isem0, isem1)
        osems = (osem0, osem1)
        ind = [
            pltpu.make_async_copy(
                x_hbm.at[src, :, pl.ds(base + j * piece, piece), :],
                bufs[j],
                isems[j],
            )
            for j in range(k)
        ]
        outd = [
            pltpu.make_async_copy(
                bufs[j],
                out_hbm.at[r, :, pl.ds(base + j * piece, piece), :],
                osems[j],
            )
            for j in range(k)
        ]
        for j in range(k):
            ind[j].start()
        for j in range(k):
            ind[j].wait()
            outd[j].start()
        for j in range(k):
            outd[j].wait()

    return gather_kernel(x)


# speedup vs baseline: 1.6864x; 1.6864x over previous
import functools

import jax
import jax.numpy as jnp
import numpy as np
from jax.experimental import pallas as pl
from jax.experimental.pallas import tpu as pltpu

_NUM_SAMPLES = 16


def _sample_indices(t: int) -> np.ndarray:
    stop = np.float32(t - 1)
    frac = np.arange(_NUM_SAMPLES - 1, dtype=np.float32) / np.float32(
        _NUM_SAMPLES - 1
    )
    vals = np.concatenate([stop * frac, np.array([stop], np.float32)])
    vals = np.clip(vals, np.float32(0.0), stop)
    return vals.astype(np.int32)


def kernel(x):
    t, c, hh, ww = x.shape
    p_total = c * hh * ww
    src = _sample_indices(t)

    sel_np = np.zeros((_NUM_SAMPLES, t), np.float32)
    sel_np[np.arange(_NUM_SAMPLES), src] = 1.0
    sel = jnp.asarray(sel_np)

    bn = 3584
    grid = p_total // bn
    assert p_total % bn == 0

    def body(sel_ref, x_ref, o_ref):
        o_ref[...] = jax.lax.dot_general(
            sel_ref[...],
            x_ref[...],
            ((( 1,), (1,)), ((), ())),
            preferred_element_type=jnp.float32,
        )

    out = pl.pallas_call(
        body,
        grid=(grid,),
        in_specs=[
            pl.BlockSpec((_NUM_SAMPLES, t), lambda n: (0, 0)),
            pl.BlockSpec((bn, t), lambda n: (n, 0)),
        ],
        out_specs=pl.BlockSpec((_NUM_SAMPLES, bn), lambda n: (0, n)),
        out_shape=jax.ShapeDtypeStruct((_NUM_SAMPLES, p_total), jnp.float32),
    )(sel, x.transpose(1, 2, 3, 0).reshape(p_total, t))

    return out.reshape(_NUM_SAMPLES, c, hh, ww)
